# Initial kernel scaffold; baseline (speedup 1.0000x reference)
import jax
import jax.numpy as jnp
from jax.experimental import pallas as pl


def kernel(x0, x1, x2, pw_W1, pw_b1, ln1_g, ln1_b, pw_W2, pw_b2, ln2_g, ln2_b, pc1_Wmlp, pc1_bmlp, pc1_Wwn, pc1_bwn, pc1_Wlin, pc1_blin, pc2_Wdown, pc2_bdown, pc2_Wmlp, pc2_bmlp, pc2_Wwn, pc2_bwn, pc2_Wlin, pc2_blin, pc2_Wup, pc2_bup, pc2_Wres, pc2_bres):
    raise NotImplementedError("write your pallas kernel here")


# speedup vs baseline: 2.9521x; 2.9521x over previous
import jax
import jax.numpy as jnp
from jax.experimental import pallas as pl
from jax.experimental.pallas import tpu as pltpu

_BIG = 1e30


def _leaky(x):
    return jnp.where(x >= 0, x, 0.2 * x)


def _expand_mats(C, W, L):
    jc = jax.lax.broadcasted_iota(jnp.int32, (C, L), 1)
    ic = jax.lax.broadcasted_iota(jnp.int32, (C, L), 0)
    E = (jc // W == ic).astype(jnp.float32)
    jw = jax.lax.broadcasted_iota(jnp.int32, (W, L), 1)
    iw = jax.lax.broadcasted_iota(jnp.int32, (W, L), 0)
    T = (jw % W == iw).astype(jnp.float32)
    return E, T


def _select_knn_accumulate(d, tcat, qc, bc, Cf, Cw, E, T, L, res_table=None):
    M, N = d.shape
    iota = jax.lax.broadcasted_iota(jnp.int32, (M, N), 1)
    agg = jnp.zeros((M, L), jnp.float32)
    res = None
    for k in range(16):
        mn = jnp.min(d, axis=1, keepdims=True)
        is_min = d == mn
        idx = jnp.min(jnp.where(is_min, iota, N), axis=1, keepdims=True)
        sel = iota == idx
        onehot = sel.astype(jnp.float32)
        d = jnp.where(sel, _BIG, d)
        if k == 0 and res_table is not None:
            res = jnp.dot(onehot, res_table, preferred_element_type=jnp.float32, precision=jax.lax.Precision.HIGHEST)
        g = jnp.dot(onehot, tcat, preferred_element_type=jnp.float32, precision=jax.lax.Precision.HIGHEST)
        fw = _leaky(g - qc + bc)
        feat = fw[:, :Cf]
        w = fw[:, Cf:Cf + Cw]
        agg = agg + jnp.dot(feat, E, preferred_element_type=jnp.float32, precision=jax.lax.Precision.HIGHEST) * jnp.dot(
            w, T, preferred_element_type=jnp.float32, precision=jax.lax.Precision.HIGHEST)
    return agg, res


def _fused_kernel(x0_ref, x1_ref, x0p_ref, x1q_ref, x2q_ref,
                  pw_W1_ref, pw_b1_ref, ln1_g_ref, ln1_b_ref,
                  pw_W2_ref, pw_b2_ref, ln2_g_ref, ln2_b_ref,
                  pc1_Wmlp_ref, pc1_bmlp_ref, pc1_Wwn_ref, pc1_bwn_ref,
                  pc1_Wlin_ref, pc1_blin_ref,
                  pc2_Wdown_ref, pc2_bdown_ref, pc2_Wmlp_ref, pc2_bmlp_ref,
                  pc2_Wwn_ref, pc2_bwn_ref, pc2_Wlin_ref, pc2_blin_ref,
                  pc2_Wup_ref, pc2_bup_ref, pc2_Wres_ref, pc2_bres_ref,
                  out_ref):
    f32 = jnp.float32
    x0r = x0_ref[0]
    x1r = x1_ref[0]
    x0p = x0p_ref[0]
    x1q = x1q_ref[0]
    x2q = x2q_ref[0]

    def ln(x, g, b):
        m = jnp.mean(x, axis=-1, keepdims=True)
        v = jnp.mean((x - m) * (x - m), axis=-1, keepdims=True)
        return (x - m) / jnp.sqrt(v + 1e-5) * g + b

    h = jnp.dot(x0p, pw_W1_ref[...], preferred_element_type=f32, precision=jax.lax.Precision.HIGHEST) + pw_b1_ref[...]
    h = _leaky(ln(h, ln1_g_ref[...], ln1_b_ref[...]))
    h = jnp.dot(h, pw_W2_ref[...], preferred_element_type=f32, precision=jax.lax.Precision.HIGHEST) + pw_b2_ref[...]
    h = _leaky(ln(h, ln2_g_ref[...], ln2_b_ref[...]))

    Wmlp1 = pc1_Wmlp_ref[...]
    Wwn1 = pc1_Wwn_ref[...]
    tW = (jnp.dot(x0p, Wmlp1[:3], preferred_element_type=f32, precision=jax.lax.Precision.HIGHEST)
          + jnp.dot(h, Wmlp1[3:], preferred_element_type=f32, precision=jax.lax.Precision.HIGHEST))
    twn = jnp.dot(x0p, Wwn1, preferred_element_type=f32, precision=jax.lax.Precision.HIGHEST)
    tcat1 = jnp.concatenate([tW, twn], axis=1)
    qc1 = jnp.concatenate(
        [jnp.dot(x1q, Wmlp1[:3], preferred_element_type=f32, precision=jax.lax.Precision.HIGHEST),
         jnp.dot(x1q, Wwn1, preferred_element_type=f32, precision=jax.lax.Precision.HIGHEST)], axis=1)
    bc1 = jnp.concatenate([pc1_bmlp_ref[...], pc1_bwn_ref[...]], axis=1)

    sq_p = x0r[0:1] * x0r[0:1] + x0r[1:2] * x0r[1:2] + x0r[2:3] * x0r[2:3]
    sq_q = (x1q[:, 0:1] * x1q[:, 0:1] + x1q[:, 1:2] * x1q[:, 1:2]
            + x1q[:, 2:3] * x1q[:, 2:3])
    d1 = sq_q + sq_p - 2.0 * jnp.dot(x1q, x0r, preferred_element_type=f32)

    E1, T1 = _expand_mats(16, 8, 128)
    agg1, _ = _select_knn_accumulate(d1, tcat1, qc1, bc1, 16, 8, E1, T1, 128)
    f1 = jnp.dot(agg1, pc1_Wlin_ref[...], preferred_element_type=f32, precision=jax.lax.Precision.HIGHEST) + pc1_blin_ref[...]

    pd = _leaky(jnp.dot(f1, pc2_Wdown_ref[...], preferred_element_type=f32, precision=jax.lax.Precision.HIGHEST)
                + pc2_bdown_ref[...])
    Wmlp2 = pc2_Wmlp_ref[...]
    Wwn2 = pc2_Wwn_ref[...]
    t2W = (jnp.dot(x1q, Wmlp2[:3], preferred_element_type=f32, precision=jax.lax.Precision.HIGHEST)
           + jnp.dot(pd, Wmlp2[3:], preferred_element_type=f32, precision=jax.lax.Precision.HIGHEST))
    t2wn = jnp.dot(x1q, Wwn2, preferred_element_type=f32, precision=jax.lax.Precision.HIGHEST)
    tcat2 = jnp.concatenate([t2W, t2wn], axis=1)
    qc2 = jnp.concatenate(
        [jnp.dot(x2q, Wmlp2[:3], preferred_element_type=f32, precision=jax.lax.Precision.HIGHEST),
         jnp.dot(x2q, Wwn2, preferred_element_type=f32, precision=jax.lax.Precision.HIGHEST)], axis=1)
    bc2 = jnp.concatenate([pc2_bmlp_ref[...], pc2_bwn_ref[...]], axis=1)

    sq_p2 = x1r[0:1] * x1r[0:1] + x1r[1:2] * x1r[1:2] + x1r[2:3] * x1r[2:3]
    sq_q2 = (x2q[:, 0:1] * x2q[:, 0:1] + x2q[:, 1:2] * x2q[:, 1:2]
             + x2q[:, 2:3] * x2q[:, 2:3])
    d2 = sq_q2 + sq_p2 - 2.0 * jnp.dot(x2q, x1r, preferred_element_type=f32)

    E2, T2 = _expand_mats(32, 8, 256)
    agg2, resg = _select_knn_accumulate(d2, tcat2, qc2, bc2, 32, 8, E2, T2, 256,
                                        res_table=f1)
    core = jnp.dot(agg2, pc2_Wlin_ref[...], preferred_element_type=f32, precision=jax.lax.Precision.HIGHEST) + pc2_blin_ref[...]
    up = jnp.dot(core, pc2_Wup_ref[...], preferred_element_type=f32, precision=jax.lax.Precision.HIGHEST) + pc2_bup_ref[...]
    res = jnp.dot(resg, pc2_Wres_ref[...], preferred_element_type=f32, precision=jax.lax.Precision.HIGHEST) + pc2_bres_ref[...]
    out_ref[0] = _leaky(up + res)


def kernel(x0, x1, x2, pw_W1, pw_b1, ln1_g, ln1_b, pw_W2, pw_b2, ln2_g, ln2_b,
           pc1_Wmlp, pc1_bmlp, pc1_Wwn, pc1_bwn, pc1_Wlin, pc1_blin,
           pc2_Wdown, pc2_bdown, pc2_Wmlp, pc2_bmlp, pc2_Wwn, pc2_bwn,
           pc2_Wlin, pc2_blin, pc2_Wup, pc2_bup, pc2_Wres, pc2_bres):
    B = x0.shape[0]
    x0p = jnp.transpose(x0, (0, 2, 1))
    x1q = jnp.transpose(x1, (0, 2, 1))
    x2q = jnp.transpose(x2, (0, 2, 1))

    row = lambda v: v.reshape(1, -1)
    weights = [pw_W1, row(pw_b1), row(ln1_g), row(ln1_b),
               pw_W2, row(pw_b2), row(ln2_g), row(ln2_b),
               pc1_Wmlp, row(pc1_bmlp), pc1_Wwn, row(pc1_bwn),
               pc1_Wlin, row(pc1_blin),
               pc2_Wdown, row(pc2_bdown), pc2_Wmlp, row(pc2_bmlp),
               pc2_Wwn, row(pc2_bwn), pc2_Wlin, row(pc2_blin),
               pc2_Wup, row(pc2_bup), pc2_Wres, row(pc2_bres)]

    batch_specs = [
        pl.BlockSpec((1,) + x0.shape[1:], lambda b: (b, 0, 0)),
        pl.BlockSpec((1,) + x1.shape[1:], lambda b: (b, 0, 0)),
        pl.BlockSpec((1,) + x0p.shape[1:], lambda b: (b, 0, 0)),
        pl.BlockSpec((1,) + x1q.shape[1:], lambda b: (b, 0, 0)),
        pl.BlockSpec((1,) + x2q.shape[1:], lambda b: (b, 0, 0)),
    ]
    weight_specs = [pl.BlockSpec(w.shape, lambda b: (0, 0)) for w in weights]

    out = pl.pallas_call(
        _fused_kernel,
        grid=(B,),
        in_specs=batch_specs + weight_specs,
        out_specs=pl.BlockSpec((1, 512, 128), lambda b: (b, 0, 0)),
        out_shape=jax.ShapeDtypeStruct((B, 512, 128), jnp.float32),
        compiler_params=pltpu.CompilerParams(
            dimension_semantics=("arbitrary",)),
    )(x0, x1, x0p, x1q, x2q, *weights)

    return jnp.transpose(out, (0, 2, 1))


# speedup vs baseline: 9.8020x; 3.3204x over previous
import jax
import jax.numpy as jnp
from jax.experimental import pallas as pl
from jax.experimental.pallas import tpu as pltpu

_BIG = 1e30


def _leaky(x):
    return jnp.where(x >= 0, x, 0.2 * x)


def _expand_mats(C, W, L):
    jc = jax.lax.broadcasted_iota(jnp.int32, (C, L), 1)
    ic = jax.lax.broadcasted_iota(jnp.int32, (C, L), 0)
    E = (jc // W == ic).astype(jnp.float32)
    jw = jax.lax.broadcasted_iota(jnp.int32, (W, L), 1)
    iw = jax.lax.broadcasted_iota(jnp.int32, (W, L), 0)
    T = (jw % W == iw).astype(jnp.float32)
    return E, T


def _hi_lo(x):
    hi = x.astype(jnp.bfloat16)
    lo = (x - hi.astype(jnp.float32)).astype(jnp.bfloat16)
    return hi, lo


def _select_knn_accumulate(d, tcat_hl, qc, bc, Cf, Cw, EE, TT, L, res_table=None):
    M, N = d.shape
    C = Cf + Cw
    iota = jax.lax.broadcasted_iota(jnp.int32, (M, N), 1)
    agg = jnp.zeros((M, L), jnp.float32)
    res = None
    for k in range(16):
        mn = jnp.min(d, axis=1, keepdims=True)
        is_min = d == mn
        idx = jnp.min(jnp.where(is_min, iota, N), axis=1, keepdims=True)
        sel = iota == idx
        oh_bf = sel.astype(jnp.bfloat16)
        d = jnp.where(sel, _BIG, d)
        if k == 0 and res_table is not None:
            res = jnp.dot(sel.astype(jnp.float32), res_table,
                          preferred_element_type=jnp.float32,
                          precision=jax.lax.Precision.HIGHEST)
        g2 = jnp.dot(oh_bf, tcat_hl, preferred_element_type=jnp.float32)
        fw = _leaky(g2[:, :C] + g2[:, C:] - qc + bc)
        fw_hi, fw_lo = _hi_lo(fw)
        feat2 = jnp.concatenate([fw_hi[:, :Cf], fw_lo[:, :Cf]], axis=1)
        w2 = jnp.concatenate([fw_hi[:, Cf:C], fw_lo[:, Cf:C]], axis=1)
        agg = agg + (jnp.dot(feat2, EE, preferred_element_type=jnp.float32)
                     * jnp.dot(w2, TT, preferred_element_type=jnp.float32))
    return agg, res


def _fused_kernel(x0_ref, x1_ref, x0p_ref, x1q_ref, x2q_ref,
                  pw_W1_ref, pw_b1_ref, ln1_g_ref, ln1_b_ref,
                  pw_W2_ref, pw_b2_ref, ln2_g_ref, ln2_b_ref,
                  pc1_Wmlp_ref, pc1_bmlp_ref, pc1_Wwn_ref, pc1_bwn_ref,
                  pc1_Wlin_ref, pc1_blin_ref,
                  pc2_Wdown_ref, pc2_bdown_ref, pc2_Wmlp_ref, pc2_bmlp_ref,
                  pc2_Wwn_ref, pc2_bwn_ref, pc2_Wlin_ref, pc2_blin_ref,
                  pc2_Wup_ref, pc2_bup_ref, pc2_Wres_ref, pc2_bres_ref,
                  out_ref):
    f32 = jnp.float32
    x0r = x0_ref[0]
    x1r = x1_ref[0]
    x0p = x0p_ref[0]
    x1q = x1q_ref[0]
    x2q = x2q_ref[0]

    def ln(x, g, b):
        m = jnp.mean(x, axis=-1, keepdims=True)
        v = jnp.mean((x - m) * (x - m), axis=-1, keepdims=True)
        return (x - m) / jnp.sqrt(v + 1e-5) * g + b

    h = jnp.dot(x0p, pw_W1_ref[...], preferred_element_type=f32, precision=jax.lax.Precision.HIGHEST) + pw_b1_ref[...]
    h = _leaky(ln(h, ln1_g_ref[...], ln1_b_ref[...]))
    h = jnp.dot(h, pw_W2_ref[...], preferred_element_type=f32, precision=jax.lax.Precision.HIGHEST) + pw_b2_ref[...]
    h = _leaky(ln(h, ln2_g_ref[...], ln2_b_ref[...]))

    Wmlp1 = pc1_Wmlp_ref[...]
    Wwn1 = pc1_Wwn_ref[...]
    tW = (jnp.dot(x0p, Wmlp1[:3], preferred_element_type=f32, precision=jax.lax.Precision.HIGHEST)
          + jnp.dot(h, Wmlp1[3:], preferred_element_type=f32, precision=jax.lax.Precision.HIGHEST))
    twn = jnp.dot(x0p, Wwn1, preferred_element_type=f32, precision=jax.lax.Precision.HIGHEST)
    tcat1 = jnp.concatenate([tW, twn], axis=1)
    qc1 = jnp.concatenate(
        [jnp.dot(x1q, Wmlp1[:3], preferred_element_type=f32, precision=jax.lax.Precision.HIGHEST),
         jnp.dot(x1q, Wwn1, preferred_element_type=f32, precision=jax.lax.Precision.HIGHEST)], axis=1)
    bc1 = jnp.concatenate([pc1_bmlp_ref[...], pc1_bwn_ref[...]], axis=1)

    sq_p = x0r[0:1] * x0r[0:1] + x0r[1:2] * x0r[1:2] + x0r[2:3] * x0r[2:3]
    sq_q = (x1q[:, 0:1] * x1q[:, 0:1] + x1q[:, 1:2] * x1q[:, 1:2]
            + x1q[:, 2:3] * x1q[:, 2:3])
    d1 = sq_q + sq_p - 2.0 * jnp.dot(x1q, x0r, preferred_element_type=f32)

    t1_hi, t1_lo = _hi_lo(tcat1)
    tcat1_hl = jnp.concatenate([t1_hi, t1_lo], axis=1)
    E1, T1 = _expand_mats(16, 8, 128)
    EE1 = jnp.concatenate([E1, E1], axis=0).astype(jnp.bfloat16)
    TT1 = jnp.concatenate([T1, T1], axis=0).astype(jnp.bfloat16)
    agg1, _ = _select_knn_accumulate(d1, tcat1_hl, qc1, bc1, 16, 8, EE1, TT1, 128)
    f1 = jnp.dot(agg1, pc1_Wlin_ref[...], preferred_element_type=f32, precision=jax.lax.Precision.HIGHEST) + pc1_blin_ref[...]

    pd = _leaky(jnp.dot(f1, pc2_Wdown_ref[...], preferred_element_type=f32, precision=jax.lax.Precision.HIGHEST)
                + pc2_bdown_ref[...])
    Wmlp2 = pc2_Wmlp_ref[...]
    Wwn2 = pc2_Wwn_ref[...]
    t2W = (jnp.dot(x1q, Wmlp2[:3], preferred_element_type=f32, precision=jax.lax.Precision.HIGHEST)
           + jnp.dot(pd, Wmlp2[3:], preferred_element_type=f32, precision=jax.lax.Precision.HIGHEST))
    t2wn = jnp.dot(x1q, Wwn2, preferred_element_type=f32, precision=jax.lax.Precision.HIGHEST)
    tcat2 = jnp.concatenate([t2W, t2wn], axis=1)
    qc2 = jnp.concatenate(
        [jnp.dot(x2q, Wmlp2[:3], preferred_element_type=f32, precision=jax.lax.Precision.HIGHEST),
         jnp.dot(x2q, Wwn2, preferred_element_type=f32, precision=jax.lax.Precision.HIGHEST)], axis=1)
    bc2 = jnp.concatenate([pc2_bmlp_ref[...], pc2_bwn_ref[...]], axis=1)

    sq_p2 = x1r[0:1] * x1r[0:1] + x1r[1:2] * x1r[1:2] + x1r[2:3] * x1r[2:3]
    sq_q2 = (x2q[:, 0:1] * x2q[:, 0:1] + x2q[:, 1:2] * x2q[:, 1:2]
             + x2q[:, 2:3] * x2q[:, 2:3])
    d2 = sq_q2 + sq_p2 - 2.0 * jnp.dot(x2q, x1r, preferred_element_type=f32)

    t2_hi, t2_lo = _hi_lo(tcat2)
    tcat2_hl = jnp.concatenate([t2_hi, t2_lo], axis=1)
    E2, T2 = _expand_mats(32, 8, 256)
    EE2 = jnp.concatenate([E2, E2], axis=0).astype(jnp.bfloat16)
    TT2 = jnp.concatenate([T2, T2], axis=0).astype(jnp.bfloat16)
    agg2, resg = _select_knn_accumulate(d2, tcat2_hl, qc2, bc2, 32, 8, EE2, TT2, 256,
                                        res_table=f1)
    core = jnp.dot(agg2, pc2_Wlin_ref[...], preferred_element_type=f32, precision=jax.lax.Precision.HIGHEST) + pc2_blin_ref[...]
    up = jnp.dot(core, pc2_Wup_ref[...], preferred_element_type=f32, precision=jax.lax.Precision.HIGHEST) + pc2_bup_ref[...]
    res = jnp.dot(resg, pc2_Wres_ref[...], preferred_element_type=f32, precision=jax.lax.Precision.HIGHEST) + pc2_bres_ref[...]
    out_ref[0] = _leaky(up + res)


def kernel(x0, x1, x2, pw_W1, pw_b1, ln1_g, ln1_b, pw_W2, pw_b2, ln2_g, ln2_b,
           pc1_Wmlp, pc1_bmlp, pc1_Wwn, pc1_bwn, pc1_Wlin, pc1_blin,
           pc2_Wdown, pc2_bdown, pc2_Wmlp, pc2_bmlp, pc2_Wwn, pc2_bwn,
           pc2_Wlin, pc2_blin, pc2_Wup, pc2_bup, pc2_Wres, pc2_bres):
    B = x0.shape[0]
    x0p = jnp.transpose(x0, (0, 2, 1))
    x1q = jnp.transpose(x1, (0, 2, 1))
    x2q = jnp.transpose(x2, (0, 2, 1))

    row = lambda v: v.reshape(1, -1)
    weights = [pw_W1, row(pw_b1), row(ln1_g), row(ln1_b),
               pw_W2, row(pw_b2), row(ln2_g), row(ln2_b),
               pc1_Wmlp, row(pc1_bmlp), pc1_Wwn, row(pc1_bwn),
               pc1_Wlin, row(pc1_blin),
               pc2_Wdown, row(pc2_bdown), pc2_Wmlp, row(pc2_bmlp),
               pc2_Wwn, row(pc2_bwn), pc2_Wlin, row(pc2_blin),
               pc2_Wup, row(pc2_bup), pc2_Wres, row(pc2_bres)]

    batch_specs = [
        pl.BlockSpec((1,) + x0.shape[1:], lambda b: (b, 0, 0)),
        pl.BlockSpec((1,) + x1.shape[1:], lambda b: (b, 0, 0)),
        pl.BlockSpec((1,) + x0p.shape[1:], lambda b: (b, 0, 0)),
        pl.BlockSpec((1,) + x1q.shape[1:], lambda b: (b, 0, 0)),
        pl.BlockSpec((1,) + x2q.shape[1:], lambda b: (b, 0, 0)),
    ]
    weight_specs = [pl.BlockSpec(w.shape, lambda b: (0, 0)) for w in weights]

    out = pl.pallas_call(
        _fused_kernel,
        grid=(B,),
        in_specs=batch_specs + weight_specs,
        out_specs=pl.BlockSpec((1, 512, 128), lambda b: (b, 0, 0)),
        out_shape=jax.ShapeDtypeStruct((B, 512, 128), jnp.float32),
        compiler_params=pltpu.CompilerParams(
            dimension_semantics=("arbitrary",)),
    )(x0, x1, x0p, x1q, x2q, *weights)

    return jnp.transpose(out, (0, 2, 1))
